# Initial kernel scaffold; baseline (speedup 1.0000x reference)
import jax
import jax.numpy as jnp
from jax.experimental import pallas as pl


def kernel(x, edge_index, edge_attr, Wq, Wk, Wv, We, Wo, w_q_norm, w_k_norm):
    raise NotImplementedError("write your pallas kernel here")


# speedup vs baseline: 1.1327x; 1.1327x over previous
import functools

import jax
import jax.numpy as jnp
import numpy as np
from jax.experimental import pallas as pl
from jax.experimental.pallas import tpu as pltpu

N = 10000
E = 320000
D = 128
H = 8
C = 16
EPS = 1e-6
QK_SCALE = 1.0 / np.sqrt(float(C))


def _qkv_body(x_ref, wq_ref, wk_ref, wv_ref, wqn_ref, wkn_ref,
              q_ref, k_ref, v_ref):
    x = x_ref[...]
    row = jax.lax.broadcasted_iota(jnp.int32, (D, H), 0)
    col = jax.lax.broadcasted_iota(jnp.int32, (D, H), 1)
    g = jnp.where(row // C == col, 1.0, 0.0).astype(jnp.float32)

    def rms(y, w_row, scale):
        y2 = y * y
        gs = jax.lax.dot(y2, g, preferred_element_type=jnp.float32)
        gsb = jax.lax.dot(gs, g.T, preferred_element_type=jnp.float32)
        return y * jax.lax.rsqrt(gsb * (1.0 / C) + EPS) * w_row * scale

    q = jax.lax.dot(x, wq_ref[...], preferred_element_type=jnp.float32)
    k = jax.lax.dot(x, wk_ref[...], preferred_element_type=jnp.float32)
    v = jax.lax.dot(x, wv_ref[...], preferred_element_type=jnp.float32)
    q_ref[...] = rms(q, wqn_ref[...], QK_SCALE)
    k_ref[...] = rms(k, wkn_ref[...], 1.0)
    v_ref[...] = v


def _qkv(x, Wq, Wk, Wv, wqn_tile, wkn_tile):
    nb = 10
    bs = N // nb
    spec_w = pl.BlockSpec((D, D), lambda i: (0, 0))
    spec_row = pl.BlockSpec((1, D), lambda i: (0, 0))
    spec_x = pl.BlockSpec((bs, D), lambda i: (i, 0))
    out = pl.pallas_call(
        _qkv_body,
        grid=(nb,),
        in_specs=[spec_x, spec_w, spec_w, spec_w, spec_row, spec_row],
        out_specs=[spec_x, spec_x, spec_x],
        out_shape=[jax.ShapeDtypeStruct((N, D), jnp.float32)] * 3,
    )(x, Wq, Wk, Wv, wqn_tile, wkn_tile)
    return out


def kernel(x, edge_index, edge_attr, Wq, Wk, Wv, We, Wo, w_q_norm, w_k_norm):
    wqn = jnp.tile(w_q_norm, D // C).reshape(1, D)
    wkn = jnp.tile(w_k_norm, D // C).reshape(1, D)
    q, k, v = _qkv(x, Wq, Wk, Wv, wqn, wkn)

    src = edge_index[0]
    dst = edge_index[1]
    e = (edge_attr @ We)
    q_dst = jnp.take(q, dst, axis=0).reshape(E, H, C)
    k_src = jnp.take(k, src, axis=0).reshape(E, H, C)
    v_src = jnp.take(v, src, axis=0).reshape(E, H, C)
    e3 = e.reshape(E, H, C)
    ke = k_src + e3
    ve = v_src + e3
    s = jnp.sum(q_dst * ke, axis=-1)
    m = jax.ops.segment_max(s, dst, num_segments=N)
    m = jnp.where(jnp.isfinite(m), m, 0.0)
    alpha = jnp.exp(s - jnp.take(m, dst, axis=0))
    denom = jax.ops.segment_sum(alpha, dst, num_segments=N)
    num = jax.ops.segment_sum(alpha[..., None] * ve, dst, num_segments=N)
    out = num / jnp.where(denom > 0, denom, 1.0)[..., None]
    out = out.reshape(N, H * C) @ Wo
    return out


# speedup vs baseline: 13.1146x; 11.5782x over previous
import functools

import jax
import jax.numpy as jnp
import numpy as np
from jax import lax
from jax.experimental import pallas as pl
from jax.experimental.pallas import tpu as pltpu
from jax.experimental.pallas import tpu_sc as plsc

N = 10000
E = 320000
D = 128
H = 8
C = 16
EPS = 1e-6
QK_SCALE = 1.0 / np.sqrt(float(C))

RW = 128
NC = 2
NS = 16
NW = NC * NS
EPW = E // NW
GE = 80
NCH = EPW // GE
NZCH = N // GE


def _qkv_body(x_ref, wq_ref, wk_ref, wv_ref, wqn_ref, wkn_ref,
              q_ref, k_ref, v_ref):
    x = x_ref[...]
    row = lax.broadcasted_iota(jnp.int32, (D, H), 0)
    col = lax.broadcasted_iota(jnp.int32, (D, H), 1)
    g = jnp.where(row // C == col, 1.0, 0.0).astype(jnp.float32)

    def rms(y, w_row, scale):
        gs = lax.dot(y * y, g, preferred_element_type=jnp.float32)
        gsb = lax.dot(gs, g.T, preferred_element_type=jnp.float32)
        return y * lax.rsqrt(gsb * (1.0 / C) + EPS) * w_row * scale

    q = lax.dot(x, wq_ref[...], preferred_element_type=jnp.float32)
    k = lax.dot(x, wk_ref[...], preferred_element_type=jnp.float32)
    v = lax.dot(x, wv_ref[...], preferred_element_type=jnp.float32)
    q_ref[...] = rms(q, wqn_ref[...], QK_SCALE)
    k_ref[...] = rms(k, wkn_ref[...], 1.0)
    v_ref[...] = v


def _qkv(x, Wq, Wk, Wv, wqn_tile, wkn_tile):
    nb = 10
    bs = N // nb
    spec_w = pl.BlockSpec((D, D), lambda i: (0, 0))
    spec_row = pl.BlockSpec((1, D), lambda i: (0, 0))
    spec_x = pl.BlockSpec((bs, D), lambda i: (i, 0))
    return pl.pallas_call(
        _qkv_body,
        grid=(nb,),
        in_specs=[spec_x, spec_w, spec_w, spec_w, spec_row, spec_row],
        out_specs=[spec_x, spec_x, spec_x],
        out_shape=[jax.ShapeDtypeStruct((N, D), jnp.float32)] * 3,
    )(x, Wq, Wk, Wv, wqn_tile, wkn_tile)


@functools.lru_cache(maxsize=None)
def _sc_kernels():
    mesh = plsc.VectorSubcoreMesh(core_axis_name="c", subcore_axis_name="s")

    @functools.partial(
        pl.kernel,
        out_type=[jax.ShapeDtypeStruct((E, D), jnp.float32)] * 3,
        mesh=mesh,
        scratch_types=[
            pltpu.VMEM((GE,), jnp.int32),
            pltpu.VMEM((GE,), jnp.int32),
            pltpu.VMEM((GE, D), jnp.float32),
            pltpu.VMEM((GE, D), jnp.float32),
            pltpu.VMEM((GE, D), jnp.float32),
            pltpu.SemaphoreType.DMA,
            pltpu.SemaphoreType.DMA,
            pltpu.SemaphoreType.DMA,
        ],
    )
    def sc_gather_qkv(q_hbm, k_hbm, v_hbm, dst_hbm, src_hbm,
                      qd_hbm, ks_hbm, vs_hbm,
                      di_v, si_v, rq_v, rk_v, rv_v, sem0, sem1, sem2):
        wid = lax.axis_index("s") * NC + lax.axis_index("c")
        base = wid * EPW

        @pl.loop(0, NCH)
        def _(ch):
            off = base + ch * GE
            pltpu.sync_copy(dst_hbm.at[pl.ds(off, GE)], di_v)
            pltpu.sync_copy(src_hbm.at[pl.ds(off, GE)], si_v)
            cq = pltpu.async_copy(q_hbm.at[di_v], rq_v, sem0)
            ck = pltpu.async_copy(k_hbm.at[si_v], rk_v, sem1)
            cv = pltpu.async_copy(v_hbm.at[si_v], rv_v, sem2)
            cq.wait()
            pltpu.sync_copy(rq_v, qd_hbm.at[pl.ds(off, GE)])
            ck.wait()
            pltpu.sync_copy(rk_v, ks_hbm.at[pl.ds(off, GE)])
            cv.wait()
            pltpu.sync_copy(rv_v, vs_hbm.at[pl.ds(off, GE)])

    @functools.partial(
        pl.kernel,
        out_type=jax.ShapeDtypeStruct((NC * N, RW), jnp.float32),
        mesh=mesh,
        scratch_types=[
            pltpu.VMEM((GE,), jnp.int32),
            pltpu.VMEM((GE, RW), jnp.float32),
            pltpu.VMEM((GE, RW), jnp.float32),
            pltpu.VMEM_SHARED((N, RW), jnp.float32),
            pltpu.SemaphoreType.DMA,
        ],
    )
    def sc_scatter_rows(rows_hbm, dst_hbm, out_hbm,
                        di_v, rows_v, zb_v, acc_sh, sem0):
        cid = lax.axis_index("c")
        sid = lax.axis_index("s")
        wid = sid * NC + cid
        base = wid * EPW

        @pl.loop(0, GE)
        def _(i):
            @pl.loop(0, RW // C)
            def _(j):
                zb_v.at[i, pl.ds(j * C, C)][...] = jnp.zeros((C,), jnp.float32)

        @pl.loop(0, NZCH)
        def _(ch):
            @pl.when(lax.rem(ch, NS) == sid)
            def _():
                pltpu.sync_copy(zb_v, acc_sh.at[pl.ds(ch * GE, GE)])

        plsc.subcore_barrier()

        @pl.loop(0, NCH)
        def _(ch):
            off = base + ch * GE
            pltpu.sync_copy(dst_hbm.at[pl.ds(off, GE)], di_v)
            pltpu.sync_copy(rows_hbm.at[pl.ds(off, GE)], rows_v)
            pltpu.sync_copy(rows_v, acc_sh.at[di_v], add=True)

        plsc.subcore_barrier()

        @pl.loop(0, NZCH)
        def _(ch):
            @pl.when(lax.rem(ch, NS) == sid)
            def _():
                pltpu.sync_copy(acc_sh.at[pl.ds(ch * GE, GE)],
                                out_hbm.at[pl.ds(cid * N + ch * GE, GE)])

    return sc_gather_qkv, sc_scatter_rows


def _edge_body(ea_ref, qd_ref, ks_ref, vs_ref, we_ref, s_ref, ve_ref):
    e = lax.dot(ea_ref[...], we_ref[...], preferred_element_type=jnp.float32)
    ke = ks_ref[...] + e
    ve_ref[...] = vs_ref[...] + e
    prod = qd_ref[...] * ke
    row = lax.broadcasted_iota(jnp.int32, (D, H), 0)
    col = lax.broadcasted_iota(jnp.int32, (D, H), 1)
    g = jnp.where(row // C == col, 1.0, 0.0).astype(jnp.float32)
    s_ref[...] = lax.dot(prod, g, preferred_element_type=jnp.float32)


def _edge_dense(ea, qd, ks, vs, We):
    bs = 4000
    nb = E // bs
    spec_e = pl.BlockSpec((bs, D), lambda i: (i, 0))
    spec_w = pl.BlockSpec((D, D), lambda i: (0, 0))
    spec_s = pl.BlockSpec((bs, H), lambda i: (i, 0))
    return pl.pallas_call(
        _edge_body,
        grid=(nb,),
        in_specs=[spec_e, spec_e, spec_e, spec_e, spec_w],
        out_specs=[spec_s, spec_e],
        out_shape=[jax.ShapeDtypeStruct((E, H), jnp.float32),
                   jax.ShapeDtypeStruct((E, D), jnp.float32)],
    )(ea, qd, ks, vs, We)


def _alpha_body(s_ref, md_ref, ve_ref, rows_ref, alpha_ref):
    alpha = jnp.exp(s_ref[...] - md_ref[...])
    row = lax.broadcasted_iota(jnp.int32, (H, D), 1)
    col = lax.broadcasted_iota(jnp.int32, (H, D), 0)
    gt = jnp.where(row // C == col, 1.0, 0.0).astype(jnp.float32)
    alpha_rep = lax.dot(alpha, gt, preferred_element_type=jnp.float32)
    rows_ref[...] = alpha_rep * ve_ref[...]
    alpha_ref[...] = alpha


def _alpha_rows(s, md, ve):
    bs = 4000
    nb = E // bs
    spec_h = pl.BlockSpec((bs, H), lambda i: (i, 0))
    return pl.pallas_call(
        _alpha_body,
        grid=(nb,),
        in_specs=[spec_h, spec_h,
                  pl.BlockSpec((bs, D), lambda i: (i, 0))],
        out_specs=[pl.BlockSpec((bs, RW), lambda i: (i, 0)), spec_h],
        out_shape=[jax.ShapeDtypeStruct((E, RW), jnp.float32),
                   jax.ShapeDtypeStruct((E, H), jnp.float32)],
    )(s, md, ve)


def _finish_body(acc_ref, den_ref, wo_ref, out_ref):
    num = acc_ref[0] + acc_ref[1]
    den = den_ref[...]
    row = lax.broadcasted_iota(jnp.int32, (H, D), 1)
    col = lax.broadcasted_iota(jnp.int32, (H, D), 0)
    gt = jnp.where(row // C == col, 1.0, 0.0).astype(jnp.float32)
    den_b = lax.dot(den, gt, preferred_element_type=jnp.float32)
    outv = num / jnp.where(den_b > 0, den_b, 1.0)
    out_ref[...] = lax.dot(outv, wo_ref[...],
                           preferred_element_type=jnp.float32)


def _finish(acc, den, Wo):
    nb = 10
    bs = N // nb
    return pl.pallas_call(
        _finish_body,
        grid=(nb,),
        in_specs=[pl.BlockSpec((NC, bs, RW), lambda i: (0, i, 0)),
                  pl.BlockSpec((bs, H), lambda i: (i, 0)),
                  pl.BlockSpec((D, D), lambda i: (0, 0))],
        out_specs=pl.BlockSpec((bs, D), lambda i: (i, 0)),
        out_shape=jax.ShapeDtypeStruct((N, D), jnp.float32),
    )(acc, den, Wo)


def kernel(x, edge_index, edge_attr, Wq, Wk, Wv, We, Wo, w_q_norm, w_k_norm):
    wqn = jnp.tile(w_q_norm, D // C).reshape(1, D)
    wkn = jnp.tile(w_k_norm, D // C).reshape(1, D)
    q, k, v = _qkv(x, Wq, Wk, Wv, wqn, wkn)

    src = edge_index[0]
    dst = edge_index[1]

    sc_gather_qkv, sc_scatter_rows = _sc_kernels()
    qd, ks, vs = sc_gather_qkv(q, k, v, dst, src)
    s, ve = _edge_dense(edge_attr, qd, ks, vs, We)

    m = jax.ops.segment_max(s, dst, num_segments=N)
    md = jnp.take(m, dst, axis=0)

    rows, alpha = _alpha_rows(s, md, ve)
    denom = jax.ops.segment_sum(alpha, dst, num_segments=N)
    acc = sc_scatter_rows(rows, dst)
    out = _finish(acc.reshape(NC, N, RW), denom, Wo)
    return out


# speedup vs baseline: 17.9712x; 1.3703x over previous
import dataclasses
import functools

import jax
import jax.numpy as jnp
import numpy as np
from jax import lax
from jax.experimental import pallas as pl
from jax.experimental.pallas import tpu as pltpu
from jax.experimental.pallas import tpu_sc as plsc

N = 10000
E = 320000
D = 128
H = 8
C = 16
EPS = 1e-6
QK_SCALE = 1.0 / np.sqrt(float(C))

RW = 128
NC = 2
NS = 16
NW = NC * NS
EPW = E // NW
GE = 80
NCH = EPW // GE
NZCH = N // GE


def _qkv_body(x_ref, wq_ref, wk_ref, wv_ref, wqn_ref, wkn_ref,
              q_ref, k_ref, v_ref):
    x = x_ref[...]
    row = lax.broadcasted_iota(jnp.int32, (D, H), 0)
    col = lax.broadcasted_iota(jnp.int32, (D, H), 1)
    g = jnp.where(row // C == col, 1.0, 0.0).astype(jnp.float32)

    def rms(y, w_row, scale):
        gs = lax.dot(y * y, g, preferred_element_type=jnp.float32)
        gsb = lax.dot(gs, g.T, preferred_element_type=jnp.float32)
        return y * lax.rsqrt(gsb * (1.0 / C) + EPS) * w_row * scale

    q = lax.dot(x, wq_ref[...], preferred_element_type=jnp.float32)
    k = lax.dot(x, wk_ref[...], preferred_element_type=jnp.float32)
    v = lax.dot(x, wv_ref[...], preferred_element_type=jnp.float32)
    q_ref[...] = rms(q, wqn_ref[...], QK_SCALE)
    k_ref[...] = rms(k, wkn_ref[...], 1.0)
    v_ref[...] = v


def _qkv(x, Wq, Wk, Wv, wqn_tile, wkn_tile):
    nb = 10
    bs = N // nb
    spec_w = pl.BlockSpec((D, D), lambda i: (0, 0))
    spec_row = pl.BlockSpec((1, D), lambda i: (0, 0))
    spec_x = pl.BlockSpec((bs, D), lambda i: (i, 0))
    return pl.pallas_call(
        _qkv_body,
        grid=(nb,),
        in_specs=[spec_x, spec_w, spec_w, spec_w, spec_row, spec_row],
        out_specs=[spec_x, spec_x, spec_x],
        out_shape=[jax.ShapeDtypeStruct((N, D), jnp.float32)] * 3,
    )(x, Wq, Wk, Wv, wqn_tile, wkn_tile)


def _md_compiler_params():
    cp = pltpu.CompilerParams()
    if "needs_layout_passes" in pltpu.CompilerParams.__dataclass_fields__:
        cp = dataclasses.replace(cp, needs_layout_passes=False)
    return cp


@functools.lru_cache(maxsize=None)
def _sc_kernels():
    mesh = plsc.VectorSubcoreMesh(core_axis_name="c", subcore_axis_name="s")

    @functools.partial(
        pl.kernel,
        out_type=[jax.ShapeDtypeStruct((E, D), jnp.float32)] * 3,
        mesh=mesh,
        scratch_types=[
            pltpu.VMEM((GE,), jnp.int32),
            pltpu.VMEM((GE,), jnp.int32),
            pltpu.VMEM((GE, D), jnp.float32),
            pltpu.VMEM((GE, D), jnp.float32),
            pltpu.VMEM((GE, D), jnp.float32),
            pltpu.SemaphoreType.DMA,
            pltpu.SemaphoreType.DMA,
            pltpu.SemaphoreType.DMA,
        ],
    )
    def sc_gather_qkv(q_hbm, k_hbm, v_hbm, dst_hbm, src_hbm,
                      qd_hbm, ks_hbm, vs_hbm,
                      di_v, si_v, rq_v, rk_v, rv_v, sem0, sem1, sem2):
        wid = lax.axis_index("s") * NC + lax.axis_index("c")
        base = wid * EPW

        @pl.loop(0, NCH)
        def _(ch):
            off = base + ch * GE
            pltpu.sync_copy(dst_hbm.at[pl.ds(off, GE)], di_v)
            pltpu.sync_copy(src_hbm.at[pl.ds(off, GE)], si_v)
            cq = pltpu.async_copy(q_hbm.at[di_v], rq_v, sem0)
            ck = pltpu.async_copy(k_hbm.at[si_v], rk_v, sem1)
            cv = pltpu.async_copy(v_hbm.at[si_v], rv_v, sem2)
            cq.wait()
            pltpu.sync_copy(rq_v, qd_hbm.at[pl.ds(off, GE)])
            ck.wait()
            pltpu.sync_copy(rk_v, ks_hbm.at[pl.ds(off, GE)])
            cv.wait()
            pltpu.sync_copy(rv_v, vs_hbm.at[pl.ds(off, GE)])

    @functools.partial(
        pl.kernel,
        out_type=jax.ShapeDtypeStruct((E * H,), jnp.float32),
        mesh=mesh,
        scratch_types=[
            pltpu.VMEM((N * H,), jnp.float32),
            pltpu.VMEM((GE,), jnp.int32),
            pltpu.VMEM((GE * H,), jnp.float32),
        ],
        compiler_params=_md_compiler_params(),
    )
    def sc_gather_md(m_hbm, dst_hbm, md_hbm, mt_v, di_v, mdbuf_v):
        wid = lax.axis_index("s") * NC + lax.axis_index("c")
        base = wid * EPW
        pltpu.sync_copy(m_hbm, mt_v)
        lane = lax.iota(jnp.int32, 16)

        @pl.loop(0, NCH)
        def _(ch):
            off = base + ch * GE
            pltpu.sync_copy(dst_hbm.at[pl.ds(off, GE)], di_v)

            @pl.loop(0, GE // 16)
            def _(g):
                dvec = di_v[pl.ds(g * 16, 16)]
                b8 = dvec * H
                wbase = lane * H + g * (16 * H)
                for h in range(H):
                    vals = plsc.load_gather(mt_v, [b8 + h])
                    plsc.store_scatter(mdbuf_v, [wbase + h], vals)

            pltpu.sync_copy(mdbuf_v, md_hbm.at[pl.ds(off * H, GE * H)])

    EPT = E // NS
    NCH2 = EPT // GE

    @functools.partial(
        pl.kernel,
        out_type=jax.ShapeDtypeStruct((NC * N, RW), jnp.float32),
        mesh=mesh,
        scratch_types=[
            pltpu.VMEM((GE,), jnp.int32),
            pltpu.VMEM((GE, RW), jnp.float32),
            pltpu.VMEM((GE, RW), jnp.float32),
            pltpu.VMEM_SHARED((N, RW), jnp.float32),
        ],
    )
    def sc_scatter_rows(rows_hbm, arep_hbm, dst_hbm, out_hbm,
                        di_v, rows_v, zb_v, acc_sh):
        cid = lax.axis_index("c")
        sid = lax.axis_index("s")

        @pl.loop(0, GE)
        def _(i):
            @pl.loop(0, RW // C)
            def _(j):
                zb_v.at[i, pl.ds(j * C, C)][...] = jnp.zeros((C,), jnp.float32)

        @pl.loop(0, NZCH)
        def _(ch):
            @pl.when(lax.rem(ch, NS) == sid)
            def _():
                pltpu.sync_copy(zb_v, acc_sh.at[pl.ds(ch * GE, GE)])

        plsc.subcore_barrier()

        base = sid * EPT

        @pl.loop(0, NCH2)
        def _(ch):
            off = base + ch * GE
            pltpu.sync_copy(dst_hbm.at[pl.ds(off, GE)], di_v)

            @pl.when(cid == 0)
            def _():
                pltpu.sync_copy(rows_hbm.at[pl.ds(off, GE)], rows_v)

            @pl.when(cid == 1)
            def _():
                pltpu.sync_copy(arep_hbm.at[pl.ds(off, GE)], rows_v)

            pltpu.sync_copy(rows_v, acc_sh.at[di_v], add=True)

        plsc.subcore_barrier()

        @pl.loop(0, NZCH)
        def _(ch):
            @pl.when(lax.rem(ch, NS) == sid)
            def _():
                pltpu.sync_copy(acc_sh.at[pl.ds(ch * GE, GE)],
                                out_hbm.at[pl.ds(cid * N + ch * GE, GE)])

    return sc_gather_qkv, sc_gather_md, sc_scatter_rows


def _edge_body(ea_ref, qd_ref, ks_ref, vs_ref, we_ref, s_ref, ve_ref):
    e = lax.dot(ea_ref[...], we_ref[...], preferred_element_type=jnp.float32)
    ke = ks_ref[...] + e
    ve_ref[...] = vs_ref[...] + e
    prod = qd_ref[...] * ke
    row = lax.broadcasted_iota(jnp.int32, (D, H), 0)
    col = lax.broadcasted_iota(jnp.int32, (D, H), 1)
    g = jnp.where(row // C == col, 1.0, 0.0).astype(jnp.float32)
    s_ref[...] = lax.dot(prod, g, preferred_element_type=jnp.float32)


def _edge_dense(ea, qd, ks, vs, We):
    bs = 4000
    nb = E // bs
    spec_e = pl.BlockSpec((bs, D), lambda i: (i, 0))
    spec_w = pl.BlockSpec((D, D), lambda i: (0, 0))
    spec_s = pl.BlockSpec((bs, H), lambda i: (i, 0))
    return pl.pallas_call(
        _edge_body,
        grid=(nb,),
        in_specs=[spec_e, spec_e, spec_e, spec_e, spec_w],
        out_specs=[spec_s, spec_e],
        out_shape=[jax.ShapeDtypeStruct((E, H), jnp.float32),
                   jax.ShapeDtypeStruct((E, D), jnp.float32)],
    )(ea, qd, ks, vs, We)


def _alpha_body(s_ref, md_ref, ve_ref, rows_ref, arep_ref):
    alpha = jnp.exp(s_ref[...] - md_ref[...])
    row = lax.broadcasted_iota(jnp.int32, (H, D), 1)
    col = lax.broadcasted_iota(jnp.int32, (H, D), 0)
    gt = jnp.where(row // C == col, 1.0, 0.0).astype(jnp.float32)
    alpha_rep = lax.dot(alpha, gt, preferred_element_type=jnp.float32)
    rows_ref[...] = alpha_rep * ve_ref[...]
    arep_ref[...] = alpha_rep


def _alpha_rows(s, md, ve):
    bs = 4000
    nb = E // bs
    spec_h = pl.BlockSpec((bs, H), lambda i: (i, 0))
    spec_d = pl.BlockSpec((bs, D), lambda i: (i, 0))
    return pl.pallas_call(
        _alpha_body,
        grid=(nb,),
        in_specs=[spec_h, spec_h, spec_d],
        out_specs=[spec_d, spec_d],
        out_shape=[jax.ShapeDtypeStruct((E, RW), jnp.float32),
                   jax.ShapeDtypeStruct((E, RW), jnp.float32)],
    )(s, md, ve)


def _finish_body(acc_ref, wo_ref, out_ref):
    num = acc_ref[0]
    den_b = acc_ref[1]
    outv = num / jnp.where(den_b > 0, den_b, 1.0)
    out_ref[...] = lax.dot(outv, wo_ref[...],
                           preferred_element_type=jnp.float32)


def _finish(acc, Wo):
    nb = 10
    bs = N // nb
    return pl.pallas_call(
        _finish_body,
        grid=(nb,),
        in_specs=[pl.BlockSpec((NC, bs, RW), lambda i: (0, i, 0)),
                  pl.BlockSpec((D, D), lambda i: (0, 0))],
        out_specs=pl.BlockSpec((bs, D), lambda i: (i, 0)),
        out_shape=jax.ShapeDtypeStruct((N, D), jnp.float32),
    )(acc, Wo)


def kernel(x, edge_index, edge_attr, Wq, Wk, Wv, We, Wo, w_q_norm, w_k_norm):
    wqn = jnp.tile(w_q_norm, D // C).reshape(1, D)
    wkn = jnp.tile(w_k_norm, D // C).reshape(1, D)
    q, k, v = _qkv(x, Wq, Wk, Wv, wqn, wkn)

    src = edge_index[0]
    dst = edge_index[1]

    sc_gather_qkv, sc_gather_md, sc_scatter_rows = _sc_kernels()
    qd, ks, vs = sc_gather_qkv(q, k, v, dst, src)
    s, ve = _edge_dense(edge_attr, qd, ks, vs, We)

    m = jax.ops.segment_max(s, dst, num_segments=N)
    md = sc_gather_md(m.reshape(N * H), dst).reshape(E, H)

    rows, arep = _alpha_rows(s, md, ve)
    acc = sc_scatter_rows(rows, arep, dst)
    out = _finish(acc.reshape(NC, N, RW), Wo)
    return out
